# baseline (speedup 1.0000x reference)
import jax
import jax.numpy as jnp
from jax.experimental import pallas as pl
from jax.experimental.pallas import tpu as pltpu
from jax.experimental.pallas import tpu_sc as plsc

_CODEBOOK = 1024
_DIM = 64
_TILE = 8192
_GATHER_WINDOW = 128


def _distance_argmin_body(z_ref, e_ref, idx_ref, acc_ref, pad_ref,
                          em2_ref, e2_ref):
    i = pl.program_id(0)

    @pl.when(i == 0)
    def _():
        e = e_ref[...]
        em2_ref[...] = -2.0 * e
        e2_ref[...] = jnp.sum(e * e, axis=1)[None, :]
        pad_ref[0, :, :_DIM] = e
        pad_ref[0, :, _DIM:] = jnp.zeros_like(e)

    z = z_ref[...]
    ze2 = jax.lax.dot_general(
        z, em2_ref[...], (((1,), (1,)), ((), ())),
        preferred_element_type=jnp.float32)
    z2 = jnp.sum(z * z, axis=1, keepdims=True)
    d = (z2 + e2_ref[...]) + ze2
    m8 = d[:, 0:128]
    for k in range(1, _CODEBOOK // 128):
        m8 = jnp.minimum(m8, d[:, k * 128:(k + 1) * 128])
    mind = jnp.min(m8, axis=1, keepdims=True)
    lane = jax.lax.broadcasted_iota(jnp.int32, d.shape, 1).astype(jnp.float32)
    idxf = jnp.min(jnp.where(d == mind, lane, jnp.float32(2 * _CODEBOOK)),
                   axis=1)
    idx_ref[0, :, 0] = idxf.astype(jnp.int32)
    acc_ref[0, 0, 0] = jnp.sum(mind)


def _gather_rows(e_pad, idx_col):
    n = idx_col.shape[0]

    @pl.kernel(
        out_type=jax.ShapeDtypeStruct((n, 128), jnp.float32),
        mesh=plsc.VectorSubcoreMesh(core_axis_name="core",
                                    subcore_axis_name="subcore"),
    )
    def sc_gather(e_hbm, i_hbm, o_hbm):
        def body(i_vmem, o_vmem):
            pltpu.sync_copy(e_hbm.at[i_vmem], o_vmem)

        pltpu.emit_pipeline(
            body,
            grid=(n // _GATHER_WINDOW,),
            in_specs=[pl.BlockSpec((_GATHER_WINDOW,),
                                   index_map=lambda i: (i,))],
            out_specs=[pl.BlockSpec((_GATHER_WINDOW, 128),
                                    index_map=lambda i: (i, 0))],
            core_axis_name=("core", "subcore"),
            dimension_semantics=(pltpu.PARALLEL,),
        )(i_hbm, o_hbm)

    return sc_gather(e_pad, idx_col)


def _repack_body(g_ref, o_ref):
    o_ref[0, 0, :, :] = g_ref[:, :_DIM]


def _repack(gathered, out_shape):
    b, s, p, dim = out_shape
    return pl.pallas_call(
        _repack_body,
        grid=(b * s,),
        in_specs=[pl.BlockSpec((p, 128), lambda i: (i, 0))],
        out_specs=pl.BlockSpec((1, 1, p, dim),
                               lambda i: (i // s, i % s, 0, 0)),
        out_shape=jax.ShapeDtypeStruct(out_shape, jnp.float32),
    )(gathered)


_CHUNKS = 1


def _distance_chunk(z_chunk, embedding_weight):
    rows, latent_dim = z_chunk.shape
    grid = rows // _TILE
    return pl.pallas_call(
        _distance_argmin_body,
        grid=(grid,),
        in_specs=[
            pl.BlockSpec((_TILE, latent_dim), lambda i: (i, 0)),
            pl.BlockSpec((_CODEBOOK, latent_dim), lambda i: (0, 0)),
        ],
        out_specs=[
            pl.BlockSpec((1, _TILE, 1), lambda i: (i, 0, 0)),
            pl.BlockSpec(memory_space=pltpu.SMEM, block_shape=(1, 1, 1),
                         index_map=lambda i: (i, 0, 0)),
            pl.BlockSpec((1, _CODEBOOK, 128), lambda i: (0, 0, 0)),
        ],
        out_shape=[
            jax.ShapeDtypeStruct((grid, _TILE, 1), jnp.int32),
            jax.ShapeDtypeStruct((grid, 1, 1), jnp.float32),
            jax.ShapeDtypeStruct((1, _CODEBOOK, 128), jnp.float32),
        ],
        scratch_shapes=[
            pltpu.VMEM((_CODEBOOK, latent_dim), jnp.float32),
            pltpu.VMEM((1, _CODEBOOK), jnp.float32),
        ],
        compiler_params=pltpu.CompilerParams(
            dimension_semantics=("arbitrary",)),
    )(z_chunk, embedding_weight)


def kernel(z, embedding_weight):
    batch_size, seq_len, num_patches, latent_dim = z.shape
    n = batch_size * seq_len * num_patches
    z_flat = z.reshape(n, latent_dim)
    rows = n // _CHUNKS

    idx_parts, zq_parts, dsums = [], [], []
    for c in range(_CHUNKS):
        z_chunk = jax.lax.slice_in_dim(z_flat, c * rows, (c + 1) * rows, axis=0)
        idx_tiles, dsum, e_pad = _distance_chunk(z_chunk, embedding_weight)
        idx_c = idx_tiles.reshape(rows)
        zq_parts.append(_gather_rows(e_pad.reshape(_CODEBOOK, 128),
                                     idx_c))
        idx_parts.append(idx_c)
        dsums.append(jnp.sum(dsum))

    idx_flat = jnp.concatenate(idx_parts)
    gathered = jnp.concatenate(zq_parts, axis=0)

    loss = 2.0 * jnp.sum(jnp.stack(dsums)) / jnp.float32(z.size)
    z_q = _repack(gathered, z.shape)
    indices = idx_flat.reshape(batch_size, seq_len, num_patches)
    return (loss, z_q, indices)


# speedup vs baseline: 1.4383x; 1.4383x over previous
import jax
import jax.numpy as jnp
from jax.experimental import pallas as pl
from jax.experimental.pallas import tpu as pltpu
from jax.experimental.pallas import tpu_sc as plsc

_CODEBOOK = 1024
_DIM = 64
_TILE = 8192
_GATHER_WINDOW = 128


def _distance_argmin_body(z_ref, e_ref, idx_ref, acc_ref, pad_ref,
                          em2_ref, e2_ref):
    i = pl.program_id(0)

    @pl.when(i == 0)
    def _():
        e = e_ref[...]
        em2_ref[...] = -2.0 * e
        e2_ref[...] = jnp.sum(e * e, axis=1)[None, :]
        pad_ref[0, :, :_DIM] = e
        pad_ref[0, :, _DIM:] = jnp.zeros_like(e)

    z = z_ref[...]
    ze2 = jax.lax.dot_general(
        z, em2_ref[...], (((1,), (1,)), ((), ())),
        preferred_element_type=jnp.float32)
    z2 = jnp.sum(z * z, axis=1, keepdims=True)
    d = (z2 + e2_ref[...]) + ze2
    m8 = d[:, 0:128]
    for k in range(1, _CODEBOOK // 128):
        m8 = jnp.minimum(m8, d[:, k * 128:(k + 1) * 128])
    mind = jnp.min(m8, axis=1, keepdims=True)
    lane = jax.lax.broadcasted_iota(jnp.int32, d.shape, 1).astype(jnp.float32)
    idxf = jnp.min(jnp.where(d == mind, lane, jnp.float32(2 * _CODEBOOK)),
                   axis=1)
    idx_ref[0, :, 0] = idxf.astype(jnp.int32)
    acc_ref[0, 0, 0] = jnp.sum(mind)


def _gather_rows(e_pad, idx_col):
    n = idx_col.shape[0]

    @pl.kernel(
        out_type=jax.ShapeDtypeStruct((n, 128), jnp.float32),
        mesh=plsc.VectorSubcoreMesh(core_axis_name="core",
                                    subcore_axis_name="subcore"),
    )
    def sc_gather(e_hbm, i_hbm, o_hbm):
        def body(i_vmem, o_vmem):
            pltpu.sync_copy(e_hbm.at[i_vmem], o_vmem)

        pltpu.emit_pipeline(
            body,
            grid=(n // _GATHER_WINDOW,),
            in_specs=[pl.BlockSpec((_GATHER_WINDOW,),
                                   index_map=lambda i: (i,))],
            out_specs=[pl.BlockSpec((_GATHER_WINDOW, 128),
                                    index_map=lambda i: (i, 0))],
            core_axis_name=("core", "subcore"),
            dimension_semantics=(pltpu.PARALLEL,),
        )(i_hbm, o_hbm)

    return sc_gather(e_pad, idx_col)


def _repack_body(g_ref, o_ref):
    s, p, dim = o_ref.shape[1:]
    o_ref[0] = g_ref[:, :_DIM].reshape(s, p, dim)


def _repack(gathered, out_shape):
    b, s, p, dim = out_shape
    return pl.pallas_call(
        _repack_body,
        grid=(b,),
        in_specs=[pl.BlockSpec((s * p, 128), lambda i: (i, 0))],
        out_specs=pl.BlockSpec((1, s, p, dim), lambda i: (i, 0, 0, 0)),
        out_shape=jax.ShapeDtypeStruct(out_shape, jnp.float32),
    )(gathered)


_CHUNKS = 1


def _distance_chunk(z_chunk, embedding_weight):
    rows, latent_dim = z_chunk.shape
    grid = rows // _TILE
    return pl.pallas_call(
        _distance_argmin_body,
        grid=(grid,),
        in_specs=[
            pl.BlockSpec((_TILE, latent_dim), lambda i: (i, 0)),
            pl.BlockSpec((_CODEBOOK, latent_dim), lambda i: (0, 0)),
        ],
        out_specs=[
            pl.BlockSpec((1, _TILE, 1), lambda i: (i, 0, 0)),
            pl.BlockSpec(memory_space=pltpu.SMEM, block_shape=(1, 1, 1),
                         index_map=lambda i: (i, 0, 0)),
            pl.BlockSpec((1, _CODEBOOK, 128), lambda i: (0, 0, 0)),
        ],
        out_shape=[
            jax.ShapeDtypeStruct((grid, _TILE, 1), jnp.int32),
            jax.ShapeDtypeStruct((grid, 1, 1), jnp.float32),
            jax.ShapeDtypeStruct((1, _CODEBOOK, 128), jnp.float32),
        ],
        scratch_shapes=[
            pltpu.VMEM((_CODEBOOK, latent_dim), jnp.float32),
            pltpu.VMEM((1, _CODEBOOK), jnp.float32),
        ],
        compiler_params=pltpu.CompilerParams(
            dimension_semantics=("arbitrary",)),
    )(z_chunk, embedding_weight)


def kernel(z, embedding_weight):
    batch_size, seq_len, num_patches, latent_dim = z.shape
    n = batch_size * seq_len * num_patches
    z_flat = z.reshape(n, latent_dim)
    rows = n // _CHUNKS

    idx_parts, zq_parts, dsums = [], [], []
    for c in range(_CHUNKS):
        z_chunk = jax.lax.slice_in_dim(z_flat, c * rows, (c + 1) * rows, axis=0)
        idx_tiles, dsum, e_pad = _distance_chunk(z_chunk, embedding_weight)
        idx_c = idx_tiles.reshape(rows)
        zq_parts.append(_gather_rows(e_pad.reshape(_CODEBOOK, 128),
                                     idx_c))
        idx_parts.append(idx_c)
        dsums.append(jnp.sum(dsum))

    idx_flat = jnp.concatenate(idx_parts)
    gathered = jnp.concatenate(zq_parts, axis=0)

    loss = 2.0 * jnp.sum(jnp.stack(dsums)) / jnp.float32(z.size)
    z_q = _repack(gathered, z.shape)
    indices = idx_flat.reshape(batch_size, seq_len, num_patches)
    return (loss, z_q, indices)


# speedup vs baseline: 1.6177x; 1.1247x over previous
import jax
import jax.numpy as jnp
from jax.experimental import pallas as pl
from jax.experimental.pallas import tpu as pltpu
from jax.experimental.pallas import tpu_sc as plsc

_CODEBOOK = 1024
_DIM = 64
_TILE = 8192
_GATHER_WINDOW = 128


def _distance_argmin_body(z_ref, e_ref, idx_ref, acc_ref, pad_ref,
                          em2_ref, e2_ref):
    i = pl.program_id(0)

    @pl.when(i == 0)
    def _():
        e = e_ref[...]
        em2_ref[...] = -2.0 * e
        e2_ref[...] = jnp.sum(e * e, axis=1)[None, :]
        pad_ref[0, :, :_DIM] = e
        pad_ref[0, :, _DIM:] = jnp.zeros_like(e)

    z = z_ref[...]
    ze2 = jax.lax.dot_general(
        z, em2_ref[...], (((1,), (1,)), ((), ())),
        preferred_element_type=jnp.float32)
    z2 = jnp.sum(z * z, axis=1, keepdims=True)
    d = (z2 + e2_ref[...]) + ze2
    m8 = d[:, 0:128]
    for k in range(1, _CODEBOOK // 128):
        m8 = jnp.minimum(m8, d[:, k * 128:(k + 1) * 128])
    mind = jnp.min(m8, axis=1, keepdims=True)
    lane = jax.lax.broadcasted_iota(jnp.int32, d.shape, 1).astype(jnp.float32)
    idxf = jnp.min(jnp.where(d == mind, lane, jnp.float32(2 * _CODEBOOK)),
                   axis=1)
    idx_ref[0, :, 0] = idxf.astype(jnp.int32)
    acc_ref[0, 0, 0] = jnp.sum(mind)


def _gather_rows(e_pad, idx_col):
    n = idx_col.shape[0]

    @pl.kernel(
        out_type=jax.ShapeDtypeStruct((n, 128), jnp.float32),
        mesh=plsc.VectorSubcoreMesh(core_axis_name="core",
                                    subcore_axis_name="subcore"),
    )
    def sc_gather(e_hbm, i_hbm, o_hbm):
        def body(i_vmem, o_vmem):
            pltpu.sync_copy(e_hbm.at[i_vmem], o_vmem)

        pltpu.emit_pipeline(
            body,
            grid=(n // _GATHER_WINDOW,),
            in_specs=[pl.BlockSpec((_GATHER_WINDOW,),
                                   index_map=lambda i: (i,))],
            out_specs=[pl.BlockSpec((_GATHER_WINDOW, 128),
                                    index_map=lambda i: (i, 0))],
            core_axis_name=("core", "subcore"),
            dimension_semantics=(pltpu.PARALLEL,),
        )(i_hbm, o_hbm)

    return sc_gather(e_pad, idx_col)


_CHUNKS = 1


def _distance_chunk(z_chunk, embedding_weight):
    rows, latent_dim = z_chunk.shape
    grid = rows // _TILE
    return pl.pallas_call(
        _distance_argmin_body,
        grid=(grid,),
        in_specs=[
            pl.BlockSpec((_TILE, latent_dim), lambda i: (i, 0)),
            pl.BlockSpec((_CODEBOOK, latent_dim), lambda i: (0, 0)),
        ],
        out_specs=[
            pl.BlockSpec((1, _TILE, 1), lambda i: (i, 0, 0)),
            pl.BlockSpec(memory_space=pltpu.SMEM, block_shape=(1, 1, 1),
                         index_map=lambda i: (i, 0, 0)),
            pl.BlockSpec((1, _CODEBOOK, 128), lambda i: (0, 0, 0)),
        ],
        out_shape=[
            jax.ShapeDtypeStruct((grid, _TILE, 1), jnp.int32),
            jax.ShapeDtypeStruct((grid, 1, 1), jnp.float32),
            jax.ShapeDtypeStruct((1, _CODEBOOK, 128), jnp.float32),
        ],
        scratch_shapes=[
            pltpu.VMEM((_CODEBOOK, latent_dim), jnp.float32),
            pltpu.VMEM((1, _CODEBOOK), jnp.float32),
        ],
        compiler_params=pltpu.CompilerParams(
            dimension_semantics=("arbitrary",)),
    )(z_chunk, embedding_weight)


def kernel(z, embedding_weight):
    batch_size, seq_len, num_patches, latent_dim = z.shape
    n = batch_size * seq_len * num_patches
    z_flat = z.reshape(n, latent_dim)
    rows = n // _CHUNKS

    idx_parts, zq_parts, dsums = [], [], []
    for c in range(_CHUNKS):
        z_chunk = jax.lax.slice_in_dim(z_flat, c * rows, (c + 1) * rows, axis=0)
        idx_tiles, dsum, e_pad = _distance_chunk(z_chunk, embedding_weight)
        idx_c = idx_tiles.reshape(rows)
        zq_parts.append(_gather_rows(e_pad.reshape(_CODEBOOK, 128),
                                     idx_c))
        idx_parts.append(idx_c)
        dsums.append(jnp.sum(dsum))

    idx_flat = jnp.concatenate(idx_parts)
    gathered = jnp.concatenate(zq_parts, axis=0)

    loss = 2.0 * jnp.sum(jnp.stack(dsums)) / jnp.float32(z.size)
    z_q = gathered[:, :_DIM].reshape(z.shape)
    indices = idx_flat.reshape(batch_size, seq_len, num_patches)
    return (loss, z_q, indices)


# speedup vs baseline: 1.6281x; 1.0064x over previous
import jax
import jax.numpy as jnp
from jax.experimental import pallas as pl
from jax.experimental.pallas import tpu as pltpu
from jax.experimental.pallas import tpu_sc as plsc

_CODEBOOK = 1024
_DIM = 64
_TILE = 8192
_GATHER_WINDOW = 256


def _distance_argmin_body(z_ref, e_ref, idx_ref, acc_ref, pad_ref,
                          em2_ref, e2_ref):
    i = pl.program_id(0)

    @pl.when(i == 0)
    def _():
        e = e_ref[...]
        em2_ref[...] = -2.0 * e
        e2_ref[...] = jnp.sum(e * e, axis=1)[None, :]
        pad_ref[0, :, :_DIM] = e
        pad_ref[0, :, _DIM:] = jnp.zeros_like(e)

    z = z_ref[...]
    ze2 = jax.lax.dot_general(
        z, em2_ref[...], (((1,), (1,)), ((), ())),
        preferred_element_type=jnp.float32)
    z2 = jnp.sum(z * z, axis=1, keepdims=True)
    d = (z2 + e2_ref[...]) + ze2
    m8 = d[:, 0:128]
    for k in range(1, _CODEBOOK // 128):
        m8 = jnp.minimum(m8, d[:, k * 128:(k + 1) * 128])
    mind = jnp.min(m8, axis=1, keepdims=True)
    lane = jax.lax.broadcasted_iota(jnp.int32, d.shape, 1).astype(jnp.float32)
    idxf = jnp.min(jnp.where(d == mind, lane, jnp.float32(2 * _CODEBOOK)),
                   axis=1)
    idx_ref[0, :, 0] = idxf.astype(jnp.int32)
    acc_ref[0, 0, 0] = jnp.sum(mind)


def _gather_rows(e_pad, idx_col):
    n = idx_col.shape[0]

    @pl.kernel(
        out_type=jax.ShapeDtypeStruct((n, 128), jnp.float32),
        mesh=plsc.VectorSubcoreMesh(core_axis_name="core",
                                    subcore_axis_name="subcore"),
    )
    def sc_gather(e_hbm, i_hbm, o_hbm):
        def body(i_vmem, o_vmem):
            pltpu.sync_copy(e_hbm.at[i_vmem], o_vmem)

        pltpu.emit_pipeline(
            body,
            grid=(n // _GATHER_WINDOW,),
            in_specs=[pl.BlockSpec((_GATHER_WINDOW,),
                                   index_map=lambda i: (i,))],
            out_specs=[pl.BlockSpec((_GATHER_WINDOW, 128),
                                    index_map=lambda i: (i, 0))],
            core_axis_name=("core", "subcore"),
            dimension_semantics=(pltpu.PARALLEL,),
        )(i_hbm, o_hbm)

    return sc_gather(e_pad, idx_col)


_CHUNKS = 1


def _distance_chunk(z_chunk, embedding_weight):
    rows, latent_dim = z_chunk.shape
    grid = rows // _TILE
    return pl.pallas_call(
        _distance_argmin_body,
        grid=(grid,),
        in_specs=[
            pl.BlockSpec((_TILE, latent_dim), lambda i: (i, 0)),
            pl.BlockSpec((_CODEBOOK, latent_dim), lambda i: (0, 0)),
        ],
        out_specs=[
            pl.BlockSpec((1, _TILE, 1), lambda i: (i, 0, 0)),
            pl.BlockSpec(memory_space=pltpu.SMEM, block_shape=(1, 1, 1),
                         index_map=lambda i: (i, 0, 0)),
            pl.BlockSpec((1, _CODEBOOK, 128), lambda i: (0, 0, 0)),
        ],
        out_shape=[
            jax.ShapeDtypeStruct((grid, _TILE, 1), jnp.int32),
            jax.ShapeDtypeStruct((grid, 1, 1), jnp.float32),
            jax.ShapeDtypeStruct((1, _CODEBOOK, 128), jnp.float32),
        ],
        scratch_shapes=[
            pltpu.VMEM((_CODEBOOK, latent_dim), jnp.float32),
            pltpu.VMEM((1, _CODEBOOK), jnp.float32),
        ],
        compiler_params=pltpu.CompilerParams(
            dimension_semantics=("arbitrary",)),
    )(z_chunk, embedding_weight)


def kernel(z, embedding_weight):
    batch_size, seq_len, num_patches, latent_dim = z.shape
    n = batch_size * seq_len * num_patches
    z_flat = z.reshape(n, latent_dim)
    rows = n // _CHUNKS

    idx_parts, zq_parts, dsums = [], [], []
    for c in range(_CHUNKS):
        z_chunk = jax.lax.slice_in_dim(z_flat, c * rows, (c + 1) * rows, axis=0)
        idx_tiles, dsum, e_pad = _distance_chunk(z_chunk, embedding_weight)
        idx_c = idx_tiles.reshape(rows)
        zq_parts.append(_gather_rows(e_pad.reshape(_CODEBOOK, 128),
                                     idx_c))
        idx_parts.append(idx_c)
        dsums.append(jnp.sum(dsum))

    idx_flat = jnp.concatenate(idx_parts)
    gathered = jnp.concatenate(zq_parts, axis=0)

    loss = 2.0 * jnp.sum(jnp.stack(dsums)) / jnp.float32(z.size)
    z_q = gathered[:, :_DIM].reshape(z.shape)
    indices = idx_flat.reshape(batch_size, seq_len, num_patches)
    return (loss, z_q, indices)
